# baseline (speedup 1.0000x reference)
import functools

import jax
import jax.numpy as jnp
from jax import lax
from jax.experimental import pallas as pl
from jax.experimental.pallas import tpu as pltpu
from jax.experimental.pallas import tpu_sc as plsc

TOKENS = 16384
E = 64
K = 8
L = 16
NC = 2
NS = 16
NW = NC * NS
ROWS = TOKENS // NW
CH = 256
NCHUNK = ROWS // CH

_mesh = plsc.VectorSubcoreMesh(core_axis_name="c", subcore_axis_name="s")


@functools.partial(
    pl.kernel,
    out_type=jax.ShapeDtypeStruct((NW, 3 * E), jnp.float32),
    mesh=_mesh,
    compiler_params=pltpu.CompilerParams(needs_layout_passes=False),
    scratch_types=[
        pltpu.VMEM((2 * CH, E), jnp.float32),
        pltpu.VMEM((3 * E,), jnp.float32),
    ],
)
def _sc_topk_hist(l_hbm, w_hbm, out_hbm, buf_v, acc_v):
    c = lax.axis_index("c")
    s = lax.axis_index("s")
    wid = s * NC + c
    base = wid * ROWS

    iota = lax.iota(jnp.int32, L)
    zeros = jnp.zeros((L,), jnp.float32)
    ones = jnp.ones((L,), jnp.float32)
    top8_mask = iota < K
    idx_consts = [iota + L * j for j in range(E // L)]
    for j in range(3 * E // L):
        acc_v[pl.ds(L * j, L)] = zeros

    def merge(ka, va, kb, vb):
        rk = lax.rev(kb, (0,))
        rv = lax.rev(vb, (0,))
        take_a = ka >= rk
        mk = jnp.maximum(ka, rk)
        mv = jnp.where(take_a, va, rv)
        return plsc.sort_key_val(mk, mv, descending=True)

    def chunk_body(chunk, carry):
        cbase = base + chunk * CH
        pltpu.sync_copy(l_hbm.at[pl.ds(cbase, CH)], buf_v.at[pl.ds(0, CH)])
        pltpu.sync_copy(w_hbm.at[pl.ds(cbase, CH)], buf_v.at[pl.ds(CH, CH)])

        @plsc.parallel_loop(0, 2 * CH, unroll=4)
        def _(r):
            is_w = r >= CH
            ks, vs = [], []
            for j in range(E // L):
                k_s, v_s = plsc.sort_key_val(
                    buf_v[r, pl.ds(L * j, L)], idx_consts[j], descending=True
                )
                ks.append(k_s)
                vs.append(v_s)
            k01, v01 = merge(ks[0], vs[0], ks[1], vs[1])
            k23, v23 = merge(ks[2], vs[2], ks[3], vs[3])
            _, vf = merge(k01, v01, k23, v23)
            off = jnp.where(is_w, jnp.int32(E), jnp.int32(0))
            plsc.addupdate_scatter(acc_v, [vf + off], ones, mask=top8_mask)

            @pl.when(is_w)
            def _():
                for j in range(E // L):
                    plsc.addupdate(
                        acc_v.at[pl.ds(2 * E + L * j, L)],
                        buf_v[r, pl.ds(L * j, L)],
                    )

        return carry

    lax.fori_loop(0, NCHUNK, chunk_body, 0)
    pltpu.sync_copy(acc_v, out_hbm.at[wid])


def _reduce_body(x_ref, o_ref):
    o_ref[...] = jnp.sum(x_ref[...], axis=0, keepdims=True)


def kernel(layer_idx, router_weights, num_experts_per_tok, router_logits):
    partials = _sc_topk_hist(router_logits, router_weights)
    out = pl.pallas_call(
        _reduce_body,
        out_shape=jax.ShapeDtypeStruct((1, 3 * E), jnp.float32),
    )(partials)
    return out.reshape(3, E)
